# Initial kernel scaffold; baseline (speedup 1.0000x reference)
import jax
import jax.numpy as jnp
from jax.experimental import pallas as pl


def kernel(x, in_proj_w, in_proj_b, out_proj_w, out_proj_b, W1, b1, ln_g, ln_b, W2, b2, W3, b3, Wt1, bt1, Wt2, bt2):
    raise NotImplementedError("write your pallas kernel here")


# speedup vs baseline: 1.9984x; 1.9984x over previous
import math

import jax
import jax.numpy as jnp
from jax import lax
from jax.experimental import pallas as pl

B, S, D, E, K, H = 4, 2048, 1024, 64, 8, 4
DH = D // H
QT = 512
KT = 1024
SB = 512
F32 = jnp.float32
cT = (((1,), (1,)), ((), ()))
cN = (((1,), (0,)), ((), ()))


def _erfc(z):
    az = jnp.abs(z)
    w = z * z
    pe = jnp.full_like(z, 7.85386146e-05)
    for c in (-0.000801019371, 0.00518832775, -0.0268538129, 0.112835854,
              -0.37612626, 1.12837911):
        pe = pe * w + jnp.float32(c)
    erf_small = 1.0 - z * pe
    q = 1.0 / w
    p1 = jnp.full_like(z, 0.0232682)
    for c in (-0.138703942, 0.368742466, -0.582473278, 0.621000469,
              -0.494451523, 0.340488, -0.274112701, 0.563825965):
        p1 = p1 * q + jnp.float32(c)
    p2 = jnp.full_like(z, -10.477664)
    for c in (12.9772, -7.49551868, 2.92101908, -1.01526523, 0.42184633,
              -0.282076746, 0.564189494):
        p2 = p2 * q + jnp.float32(c)
    pp = jnp.where(az < 2.0, p1, p2)
    r = (jnp.exp(-w) * (1.0 / az)) * pp
    r = jnp.where(-w < -88.7228394, 0.0, r)
    r = jnp.where(z < 0.0, 2.0 - r, r)
    return jnp.where(az < 1.0, erf_small, r)


def _gelu(x):
    return (x * 0.5) * _erfc(-x * 0.707106769)


def _dot(a, b, cdims):
    return lax.dot_general(a, b, (cdims, ((), ())),
                           preferred_element_type=F32)


def qkv_body(x_ref, w_ref, b_ref, o_ref):
    o_ref[0] = _dot(x_ref[0], w_ref[...], cT[0]) + b_ref[...]


def attn_body(q_ref, k_ref, v_ref, o_ref):
    def qstep(qt, carry):
        qb = q_ref[0, pl.ds(qt * QT, QT), :].astype(jnp.bfloat16)
        m = jnp.full((QT, 1), -jnp.inf, F32)
        z = jnp.zeros((QT, 1), F32)
        on = jnp.zeros((QT, DH), F32)
        for kk in range(S // KT):
            kb = k_ref[0, kk * KT:(kk + 1) * KT, :].astype(jnp.bfloat16)
            s = _dot(qb, kb, cT[0]) * (1.0 / 16.0)
            mt = jnp.max(s, axis=1, keepdims=True)
            mn = jnp.maximum(m, mt)
            delta = jnp.where(m == mn, 0.0, m - mn)
            ed = jnp.exp(delta)
            p = jnp.exp(s - mn)
            w = ed * z
            zn = w + jnp.sum(p, axis=1, keepdims=True)
            raw = on * w + _dot(p, v_ref[0, kk * KT:(kk + 1) * KT, :], cN[0])
            on = raw * (1.0 / zn)
            m, z = mn, zn
        o_ref[0, pl.ds(qt * QT, QT), :] = on.astype(jnp.bfloat16)
        return carry

    lax.fori_loop(0, S // QT, qstep, 0)


def pool_body(o_ref, x_ref, ow_ref, ob_ref, acc_ref):
    ns = pl.program_id(1)
    val = _dot(o_ref[0], ow_ref[...].astype(jnp.bfloat16), cT[0])
    val = val + ob_ref[...] + x_ref[0]
    part = jnp.sum(val, axis=0, keepdims=True)

    @pl.when(ns == 0)
    def _():
        acc_ref[0] = part

    @pl.when(ns != 0)
    def _():
        acc_ref[0] = acc_ref[0] + part


def gate_head_body(cs_ref, w1_ref, b1_ref, g_ref, be_ref, w2_ref, b2_ref,
                   w3_ref, b3_ref, wt1_ref, bt1_ref, wt2_ref, bt2_ref,
                   gates_ref, idx_ref, logits_ref, loss_ref):
    pooled = cs_ref[...] * (1.0 / S)

    h1 = _gelu(_dot(pooled, w1_ref[...], cT[0]) + b1_ref[...])
    mu = jnp.mean(h1, axis=1, keepdims=True)
    var = jnp.mean((h1 - mu) ** 2, axis=1, keepdims=True)
    h1 = (h1 - mu) / jnp.sqrt(var + 1e-5) * g_ref[...] + be_ref[...]
    h2 = _gelu(_dot(h1, w2_ref[...], cT[0]) + b2_ref[...])
    logits = _dot(h2, w3_ref[...], cT[0]) + b3_ref[...]

    t1 = _gelu(_dot(pooled, wt1_ref[...], cT[0]) + bt1_ref[...])
    t = jnp.sum(t1 * wt2_ref[...], axis=1, keepdims=True) + bt2_ref[...]
    t = jnp.maximum(t, 0.0) + jnp.log1p(jnp.exp(-jnp.abs(t)))
    t = jnp.clip(t, 0.1, 5.0)
    logits = logits / t
    logits_ref[...] = logits

    gm = jnp.max(logits, axis=1, keepdims=True)
    ge = jnp.exp(logits - gm)
    gp = ge / jnp.sum(ge, axis=1, keepdims=True)
    ep = jnp.mean(gp, axis=0, keepdims=True)
    em = jnp.mean(ep)
    var_loss = jnp.sum((ep - em) ** 2, axis=1, keepdims=True) / (E - 1) * E
    ent = -jnp.sum(ep * jnp.log(ep + 1e-8), axis=1, keepdims=True)
    loss_ref[...] = var_loss + 0.1 * (math.log(E) - ent)

    l = logits
    iota = lax.broadcasted_iota(jnp.int32, (B, E), 1)
    vals, idxs = [], []
    for _ in range(K):
        m = jnp.max(l, axis=1, keepdims=True)
        ii = jnp.min(jnp.where(l == m, iota, E), axis=1, keepdims=True)
        vals.append(m)
        idxs.append(ii)
        l = jnp.where(iota == ii, -3.4e38, l)
    vals = jnp.concatenate(vals, axis=1)
    idx = jnp.concatenate(idxs, axis=1)
    e = jnp.exp(vals - jnp.max(vals, axis=1, keepdims=True))
    gates_ref[...] = e / jnp.sum(e, axis=1, keepdims=True)
    idx_ref[...] = idx


def _qkv(x, in_proj_w, in_proj_b):
    return pl.pallas_call(
        qkv_body,
        grid=(B, 4),
        in_specs=[
            pl.BlockSpec((1, S // 4, D), lambda b, m: (b, m, 0)),
            pl.BlockSpec((3 * D, D), lambda b, m: (0, 0)),
            pl.BlockSpec((1, 3 * D), lambda b, m: (0, 0)),
        ],
        out_specs=pl.BlockSpec((1, S // 4, 3 * D), lambda b, m: (b, m, 0)),
        out_shape=jax.ShapeDtypeStruct((B, S, 3 * D), F32),
    )(x, in_proj_w, in_proj_b.reshape(1, 3 * D))


def _attn(qkv):
    return pl.pallas_call(
        attn_body,
        grid=(B, H),
        in_specs=[
            pl.BlockSpec((1, S, DH), lambda b, h: (b, 0, h)),
            pl.BlockSpec((1, S, DH), lambda b, h: (b, 0, H + h)),
            pl.BlockSpec((1, S, DH), lambda b, h: (b, 0, 2 * H + h)),
        ],
        out_specs=pl.BlockSpec((1, S, DH), lambda b, h: (b, 0, h)),
        out_shape=jax.ShapeDtypeStruct((B, S, D), jnp.bfloat16),
    )(qkv, qkv, qkv)


def _pool(o, x, out_proj_w, out_proj_b):
    return pl.pallas_call(
        pool_body,
        grid=(B, S // SB),
        in_specs=[
            pl.BlockSpec((1, SB, D), lambda b, ns: (b, ns, 0)),
            pl.BlockSpec((1, SB, D), lambda b, ns: (b, ns, 0)),
            pl.BlockSpec((D, D), lambda b, ns: (0, 0)),
            pl.BlockSpec((1, D), lambda b, ns: (0, 0)),
        ],
        out_specs=pl.BlockSpec((1, 1, D), lambda b, ns: (b, 0, 0)),
        out_shape=jax.ShapeDtypeStruct((B, 1, D), F32),
    )(o, x, out_proj_w, out_proj_b.reshape(1, D))


def _gate_head(cs, W1, b1, ln_g, ln_b, W2, b2, W3, b3, Wt1, bt1, Wt2, bt2):
    return pl.pallas_call(
        gate_head_body,
        out_shape=[
            jax.ShapeDtypeStruct((B, K), F32),
            jax.ShapeDtypeStruct((B, K), jnp.int32),
            jax.ShapeDtypeStruct((B, E), F32),
            jax.ShapeDtypeStruct((1, 1), F32),
        ],
    )(cs, W1, b1.reshape(1, D), ln_g.reshape(1, D), ln_b.reshape(1, D),
      W2, b2.reshape(1, D // 2), W3, b3.reshape(1, E), Wt1,
      bt1.reshape(1, D // 4), Wt2, bt2.reshape(1, 1))


def kernel(x, in_proj_w, in_proj_b, out_proj_w, out_proj_b, W1, b1, ln_g,
           ln_b, W2, b2, W3, b3, Wt1, bt1, Wt2, bt2):
    qkv = _qkv(x, in_proj_w, in_proj_b)
    o = _attn(qkv)
    cs = _pool(o, x, out_proj_w, out_proj_b).reshape(B, D)
    gates, idx, _logits, loss = _gate_head(cs, W1, b1, ln_g, ln_b, W2, b2,
                                           W3, b3, Wt1, bt1, Wt2, bt2)
    return gates, idx, loss.reshape(())
